# baseline (speedup 1.0000x reference)
import functools

import jax
import jax.numpy as jnp
from jax import lax
from jax.experimental import pallas as pl
from jax.experimental.pallas import tpu as pltpu
from jax.experimental.pallas import tpu_sc as plsc

E = 320000
D = 128
N = 10000
NC = 2
NS = 16
NW = NC * NS
EW = E // NW
B = 100
K = EW // B
RZ = 640
NP = NS * RZ
CW = 8
ZB = 80
NZ = RZ // ZB

_mesh = plsc.VectorSubcoreMesh(
    core_axis_name="c", subcore_axis_name="s", num_cores=NC, num_subcores=NS
)


@functools.partial(
    pl.kernel,
    out_type=(
        jax.ShapeDtypeStruct((NC, NP, D), jnp.float32),
        jax.ShapeDtypeStruct((NC, NP, CW), jnp.float32),
    ),
    mesh=_mesh,
    compiler_params=pltpu.CompilerParams(use_tc_tiling_on_sc=False),
    scratch_types=[
        pltpu.VMEM((B, D), jnp.float32),
        pltpu.VMEM((B, D), jnp.float32),
        pltpu.VMEM((B, D), jnp.float32),
        pltpu.VMEM((1, B), jnp.int32),
        pltpu.VMEM((1, B), jnp.int32),
        pltpu.VMEM((1, B), jnp.int32),
        pltpu.VMEM((B, CW), jnp.float32),
        pltpu.VMEM((ZB, CW), jnp.float32),
        pltpu.VMEM((ZB, CW), jnp.float32),
        pltpu.SemaphoreType.DMA,
        pltpu.SemaphoreType.DMA,
        pltpu.SemaphoreType.DMA,
        pltpu.SemaphoreType.DMA,
        pltpu.SemaphoreType.DMA,
        pltpu.SemaphoreType.DMA,
        pltpu.SemaphoreType.DMA,
        pltpu.SemaphoreType.DMA,
        pltpu.SemaphoreType.DMA,
        pltpu.VMEM_SHARED((NP, D), jnp.float32),
        pltpu.VMEM_SHARED((NP, CW), jnp.float32),
    ],
)
def _sc_partials(x_hbm, idx_hbm, ones_hbm, zs_hbm, zc_hbm,
                 ps_hbm, pc_hbm, xbuf0, xbuf1, xbuf2, ibuf0, ibuf1, ibuf2,
                 obuf, cbuf0, cbuf1, lsem0, lsem1, lsem2,
                 ssem0, ssem1, ssem2, csem, asemA, asemB, acc, cnt):
    c = lax.axis_index("c")
    s = lax.axis_index("s")
    wid = c * NS + s
    pltpu.sync_copy(zs_hbm, xbuf0.at[pl.ds(0, ZB)])
    pltpu.sync_copy(zc_hbm, cbuf0)
    for j in range(NZ):
        pltpu.async_copy(xbuf0.at[pl.ds(0, ZB)],
                         acc.at[pl.ds(s * RZ + j * ZB, ZB)], asemA)
        pltpu.async_copy(cbuf0, cnt.at[pl.ds(s * RZ + j * ZB, ZB)], asemB)
    pltpu.sync_copy(ones_hbm, obuf)
    for j in range(NZ):
        pltpu.make_async_copy(xbuf0.at[pl.ds(0, ZB)],
                              acc.at[pl.ds(s * RZ + j * ZB, ZB)], asemA).wait()
        pltpu.make_async_copy(cbuf0, cnt.at[pl.ds(s * RZ + j * ZB, ZB)],
                              asemB).wait()
    plsc.subcore_barrier()

    ebase = wid * EW
    bufs = (xbuf0, xbuf1, xbuf2)
    ibufs = (ibuf0, ibuf1, ibuf2)
    lsems = (lsem0, lsem1, lsem2)
    ssems = (ssem0, ssem1, ssem2)

    def fire_load(k, b):
        pltpu.async_copy(x_hbm.at[pl.ds(ebase + k * B, B)], bufs[b], lsems[b])
        pltpu.async_copy(idx_hbm.at[wid, k], ibufs[b], lsems[b])

    def wait_load(k, b):
        pltpu.make_async_copy(x_hbm.at[pl.ds(ebase + k * B, B)], bufs[b],
                              lsems[b]).wait()
        pltpu.make_async_copy(idx_hbm.at[wid, k], ibufs[b], lsems[b]).wait()

    def fire_sc(b):
        pltpu.async_copy(bufs[b], acc.at[ibufs[b].at[0]], ssems[b], add=True)

    def wait_sc(b):
        pltpu.make_async_copy(bufs[b], acc.at[ibufs[b].at[0]], ssems[b]).wait()

    def fire_cnt(b):
        pltpu.async_copy(obuf, cnt.at[ibufs[b].at[0]], csem, add=True)

    def wait_cnt(b):
        pltpu.make_async_copy(obuf, cnt.at[ibufs[b].at[0]], csem).wait()

    fire_load(0, 0)
    fire_load(1, 1)
    fire_load(2, 2)
    wait_load(0, 0); fire_sc(0); fire_cnt(0)
    wait_load(1, 1); fire_sc(1); wait_cnt(0); fire_cnt(1)
    wait_sc(0); fire_load(3, 0)
    wait_load(2, 2); fire_sc(2); wait_cnt(1); fire_cnt(2)
    wait_sc(1); fire_load(4, 1)

    def group(g, carry):
        for b3 in range(3):
            k = 3 * g + 3 + b3
            b = b3
            bp = (b3 + 2) % 3
            wait_load(k, b)
            fire_sc(b)
            wait_cnt(bp)
            fire_cnt(b)
            wait_sc(bp)
            fire_load(k + 2, bp)
        return carry

    lax.fori_loop(0, (K - 5) // 3, group, 0)

    for k in (K - 4, K - 3):
        b = k % 3
        bp = (b + 2) % 3
        wait_load(k, b)
        fire_sc(b)
        wait_cnt(bp)
        fire_cnt(b)
        wait_sc(bp)
        fire_load(k + 2, bp)
    for k in (K - 2, K - 1):
        b = k % 3
        bp = (b + 2) % 3
        wait_load(k, b)
        fire_sc(b)
        wait_cnt(bp)
        fire_cnt(b)
        wait_sc(bp)
    wait_sc((K - 1) % 3)
    wait_cnt((K - 1) % 3)

    plsc.subcore_barrier()

    cbufs = (cbuf0, cbuf1)

    def rd(j, b):
        off = s * RZ + j * ZB
        pltpu.async_copy(acc.at[pl.ds(off, ZB)], bufs[b].at[pl.ds(0, ZB)], asemA)
        pltpu.async_copy(cnt.at[pl.ds(off, ZB)], cbufs[b], asemB)

    def wait_rd(j, b):
        off = s * RZ + j * ZB
        pltpu.make_async_copy(acc.at[pl.ds(off, ZB)], bufs[b].at[pl.ds(0, ZB)],
                              asemA).wait()
        pltpu.make_async_copy(cnt.at[pl.ds(off, ZB)], cbufs[b], asemB).wait()

    def wr(j, b):
        off = s * RZ + j * ZB
        pltpu.async_copy(bufs[b].at[pl.ds(0, ZB)], ps_hbm.at[c, pl.ds(off, ZB)],
                         lsems[b])
        pltpu.async_copy(cbufs[b], pc_hbm.at[c, pl.ds(off, ZB)], csem)

    def wait_wr(j, b):
        off = s * RZ + j * ZB
        pltpu.make_async_copy(bufs[b].at[pl.ds(0, ZB)],
                              ps_hbm.at[c, pl.ds(off, ZB)], lsems[b]).wait()
        pltpu.make_async_copy(cbufs[b], pc_hbm.at[c, pl.ds(off, ZB)],
                              csem).wait()

    rd(0, 0)
    wait_rd(0, 0)
    wr(0, 0)
    rd(1, 1)
    for j in range(1, NZ):
        b = j % 2
        wait_rd(j, b)
        wr(j, b)
        wait_wr(j - 1, 1 - b)
        if j + 1 < NZ:
            rd(j + 1, 1 - b)
    wait_wr(NZ - 1, (NZ - 1) % 2)


ROWS_BLK = 400


def _combine_body(ps_ref, pc_ref, o_ref):
    ssum = ps_ref[0] + ps_ref[1]
    csum = pc_ref[0] + pc_ref[1]
    o_ref[...] = ssum / jnp.maximum(csum[:, 0:1], 1.0)


_combine = pl.pallas_call(
    _combine_body,
    grid=(N // ROWS_BLK,),
    in_specs=[
        pl.BlockSpec((NC, ROWS_BLK, D), lambda i: (0, i, 0)),
        pl.BlockSpec((NC, ROWS_BLK, CW), lambda i: (0, i, 0)),
    ],
    out_specs=pl.BlockSpec((ROWS_BLK, D), lambda i: (i, 0)),
    out_shape=jax.ShapeDtypeStruct((N, D), jnp.float32),
)


def kernel(x, index, dim_size):
    del dim_size
    idx4d = index.astype(jnp.int32).reshape(NW, K, 1, B)
    ones = jnp.ones((B, CW), jnp.float32)
    zs = jnp.zeros((ZB, D), jnp.float32)
    zc = jnp.zeros((ZB, CW), jnp.float32)
    ps, pc = _sc_partials(x, idx4d, ones, zs, zc)
    return _combine(ps, pc)


# speedup vs baseline: 1.0330x; 1.0330x over previous
import functools

import jax
import jax.numpy as jnp
from jax import lax
from jax.experimental import pallas as pl
from jax.experimental.pallas import tpu as pltpu
from jax.experimental.pallas import tpu_sc as plsc

E = 320000
D = 128
N = 10000
NC = 2
NS = 16
NW = NC * NS
EW = E // NW
B = 125
K = EW // B
RZ = 640
NP = NS * RZ
CW = 8
ZB = 80
NZ = RZ // ZB

_mesh = plsc.VectorSubcoreMesh(
    core_axis_name="c", subcore_axis_name="s", num_cores=NC, num_subcores=NS
)


@functools.partial(
    pl.kernel,
    out_type=(
        jax.ShapeDtypeStruct((NC, NP, D), jnp.float32),
        jax.ShapeDtypeStruct((NC, NP, CW), jnp.float32),
    ),
    mesh=_mesh,
    compiler_params=pltpu.CompilerParams(use_tc_tiling_on_sc=False),
    scratch_types=[
        pltpu.VMEM((B, D), jnp.float32),
        pltpu.VMEM((B, D), jnp.float32),
        pltpu.VMEM((1, B), jnp.int32),
        pltpu.VMEM((1, B), jnp.int32),
        pltpu.VMEM((B, CW), jnp.float32),
        pltpu.VMEM((ZB, CW), jnp.float32),
        pltpu.VMEM((ZB, CW), jnp.float32),
        pltpu.SemaphoreType.DMA,
        pltpu.SemaphoreType.DMA,
        pltpu.SemaphoreType.DMA,
        pltpu.SemaphoreType.DMA,
        pltpu.SemaphoreType.DMA,
        pltpu.VMEM_SHARED((NP, D), jnp.float32),
        pltpu.VMEM_SHARED((NP, CW), jnp.float32),
    ],
)
def _sc_partials(x_hbm, idx_hbm, ones_hbm, zs_hbm, zc_hbm,
                 ps_hbm, pc_hbm, xbuf0, xbuf1, ibuf0, ibuf1,
                 obuf, cbuf0, cbuf1, lsem0, lsem1, csem, asemA, asemB,
                 acc, cnt):
    c = lax.axis_index("c")
    s = lax.axis_index("s")
    wid = c * NS + s
    pltpu.sync_copy(zs_hbm, xbuf0.at[pl.ds(0, ZB)])
    pltpu.sync_copy(zc_hbm, cbuf0)
    for j in range(NZ):
        pltpu.async_copy(xbuf0.at[pl.ds(0, ZB)],
                         acc.at[pl.ds(s * RZ + j * ZB, ZB)], asemA)
        pltpu.async_copy(cbuf0, cnt.at[pl.ds(s * RZ + j * ZB, ZB)], asemB)
    pltpu.sync_copy(ones_hbm, obuf)
    for j in range(NZ):
        pltpu.make_async_copy(xbuf0.at[pl.ds(0, ZB)],
                              acc.at[pl.ds(s * RZ + j * ZB, ZB)], asemA).wait()
        pltpu.make_async_copy(cbuf0, cnt.at[pl.ds(s * RZ + j * ZB, ZB)],
                              asemB).wait()
    plsc.subcore_barrier()

    ebase = wid * EW
    bufs = (xbuf0, xbuf1)
    ibufs = (ibuf0, ibuf1)
    lsems = (lsem0, lsem1)

    def fire_load(k, b):
        pltpu.async_copy(x_hbm.at[pl.ds(ebase + k * B, B)], bufs[b], lsems[b])
        pltpu.async_copy(idx_hbm.at[wid, k], ibufs[b], lsems[b])

    def wait_load(k, b):
        pltpu.make_async_copy(x_hbm.at[pl.ds(ebase + k * B, B)], bufs[b],
                              lsems[b]).wait()
        pltpu.make_async_copy(idx_hbm.at[wid, k], ibufs[b], lsems[b]).wait()

    def fire_cnt(b):
        pltpu.async_copy(obuf, cnt.at[ibufs[b].at[0]], csem, add=True)

    def wait_cnt(b):
        pltpu.make_async_copy(obuf, cnt.at[ibufs[b].at[0]], csem).wait()

    fire_load(0, 0)
    fire_load(1, 1)
    wait_load(0, 0)
    pltpu.sync_copy(xbuf0, acc.at[ibuf0.at[0]], add=True)
    fire_cnt(0)
    fire_load(2, 0)

    def group(g, carry):
        for b, dk in ((1, 1), (0, 2)):
            k = 2 * g + dk
            wait_load(k, b)
            pltpu.sync_copy(bufs[b], acc.at[ibufs[b].at[0]], add=True)
            wait_cnt(1 - b)
            fire_cnt(b)
            fire_load(k + 2, b)
        return carry

    lax.fori_loop(0, K // 2 - 2, group, 0)

    k = K - 3
    wait_load(k, 1)
    pltpu.sync_copy(xbuf1, acc.at[ibuf1.at[0]], add=True)
    wait_cnt(0)
    fire_cnt(1)
    fire_load(K - 1, 1)
    for k in (K - 2, K - 1):
        b = k % 2
        wait_load(k, b)
        pltpu.sync_copy(bufs[b], acc.at[ibufs[b].at[0]], add=True)
        wait_cnt(1 - b)
        fire_cnt(b)
    wait_cnt((K - 1) % 2)

    plsc.subcore_barrier()

    cbufs = (cbuf0, cbuf1)

    def rd(j, b):
        off = s * RZ + j * ZB
        pltpu.async_copy(acc.at[pl.ds(off, ZB)], bufs[b].at[pl.ds(0, ZB)], asemA)
        pltpu.async_copy(cnt.at[pl.ds(off, ZB)], cbufs[b], asemB)

    def wait_rd(j, b):
        off = s * RZ + j * ZB
        pltpu.make_async_copy(acc.at[pl.ds(off, ZB)], bufs[b].at[pl.ds(0, ZB)],
                              asemA).wait()
        pltpu.make_async_copy(cnt.at[pl.ds(off, ZB)], cbufs[b], asemB).wait()

    def wr(j, b):
        off = s * RZ + j * ZB
        pltpu.async_copy(bufs[b].at[pl.ds(0, ZB)], ps_hbm.at[c, pl.ds(off, ZB)],
                         lsems[b])
        pltpu.async_copy(cbufs[b], pc_hbm.at[c, pl.ds(off, ZB)], csem)

    def wait_wr(j, b):
        off = s * RZ + j * ZB
        pltpu.make_async_copy(bufs[b].at[pl.ds(0, ZB)],
                              ps_hbm.at[c, pl.ds(off, ZB)], lsems[b]).wait()
        pltpu.make_async_copy(cbufs[b], pc_hbm.at[c, pl.ds(off, ZB)],
                              csem).wait()

    rd(0, 0)
    wait_rd(0, 0)
    wr(0, 0)
    rd(1, 1)
    for j in range(1, NZ):
        b = j % 2
        wait_rd(j, b)
        wr(j, b)
        wait_wr(j - 1, 1 - b)
        if j + 1 < NZ:
            rd(j + 1, 1 - b)
    wait_wr(NZ - 1, (NZ - 1) % 2)


ROWS_BLK = 400


def _combine_body(ps_ref, pc_ref, o_ref):
    ssum = ps_ref[0] + ps_ref[1]
    csum = pc_ref[0] + pc_ref[1]
    o_ref[...] = ssum / jnp.maximum(csum[:, 0:1], 1.0)


_combine = pl.pallas_call(
    _combine_body,
    grid=(N // ROWS_BLK,),
    in_specs=[
        pl.BlockSpec((NC, ROWS_BLK, D), lambda i: (0, i, 0)),
        pl.BlockSpec((NC, ROWS_BLK, CW), lambda i: (0, i, 0)),
    ],
    out_specs=pl.BlockSpec((ROWS_BLK, D), lambda i: (i, 0)),
    out_shape=jax.ShapeDtypeStruct((N, D), jnp.float32),
)


def kernel(x, index, dim_size):
    del dim_size
    idx4d = index.astype(jnp.int32).reshape(NW, K, 1, B)
    ones = jnp.ones((B, CW), jnp.float32)
    zs = jnp.zeros((ZB, D), jnp.float32)
    zc = jnp.zeros((ZB, CW), jnp.float32)
    ps, pc = _sc_partials(x, idx4d, ones, zs, zc)
    return _combine(ps, pc)


# speedup vs baseline: 1.0398x; 1.0066x over previous
import functools

import jax
import jax.numpy as jnp
from jax import lax
from jax.experimental import pallas as pl
from jax.experimental.pallas import tpu as pltpu
from jax.experimental.pallas import tpu_sc as plsc

E = 320000
D = 128
N = 10000
NC = 2
NS = 16
HW = D // NC
EW = E // NS
B = 125
K = EW // B
RZ = 640
NP = NS * RZ
CW = 16
ZB = 80
NZ = RZ // ZB
NZV = (N - (NS - 1) * RZ) // ZB

_mesh = plsc.VectorSubcoreMesh(
    core_axis_name="c", subcore_axis_name="s", num_cores=NC, num_subcores=NS
)


@functools.partial(
    pl.kernel,
    out_type=jax.ShapeDtypeStruct((N, D), jnp.float32),
    mesh=_mesh,
    compiler_params=pltpu.CompilerParams(use_tc_tiling_on_sc=False),
    scratch_types=[
        pltpu.VMEM((B, HW), jnp.float32),
        pltpu.VMEM((B, HW), jnp.float32),
        pltpu.VMEM((1, B), jnp.int32),
        pltpu.VMEM((1, B), jnp.int32),
        pltpu.VMEM((B, CW), jnp.float32),
        pltpu.VMEM((ZB, CW), jnp.float32),
        pltpu.SemaphoreType.DMA,
        pltpu.SemaphoreType.DMA,
        pltpu.SemaphoreType.DMA,
        pltpu.SemaphoreType.DMA,
        pltpu.SemaphoreType.DMA,
        pltpu.VMEM_SHARED((NP, HW), jnp.float32),
        pltpu.VMEM_SHARED((NP, CW), jnp.float32),
    ],
)
def _sc_mean(x_hbm, idx_hbm, ones_hbm, zs_hbm, zc_hbm, out_hbm,
             xbuf0, xbuf1, ibuf0, ibuf1, obuf, cbuf,
             lsem0, lsem1, csem, asemA, asemB, acc, cnt):
    c = lax.axis_index("c")
    s = lax.axis_index("s")
    pltpu.sync_copy(zs_hbm, xbuf0.at[pl.ds(0, ZB)])
    pltpu.sync_copy(zc_hbm, cbuf)
    for j in range(NZ):
        pltpu.async_copy(xbuf0.at[pl.ds(0, ZB)],
                         acc.at[pl.ds(s * RZ + j * ZB, ZB)], asemA)
        pltpu.async_copy(cbuf, cnt.at[pl.ds(s * RZ + j * ZB, ZB)], asemB)
    pltpu.sync_copy(ones_hbm, obuf)
    for j in range(NZ):
        pltpu.make_async_copy(xbuf0.at[pl.ds(0, ZB)],
                              acc.at[pl.ds(s * RZ + j * ZB, ZB)], asemA).wait()
        pltpu.make_async_copy(cbuf, cnt.at[pl.ds(s * RZ + j * ZB, ZB)],
                              asemB).wait()
    plsc.subcore_barrier()

    ebase = s * EW
    bufs = (xbuf0, xbuf1)
    ibufs = (ibuf0, ibuf1)
    lsems = (lsem0, lsem1)

    def fire_load(k, b):
        pltpu.async_copy(
            x_hbm.at[pl.ds(ebase + k * B, B), pl.ds(c * HW, HW)], bufs[b],
            lsems[b])
        pltpu.async_copy(idx_hbm.at[s, k], ibufs[b], lsems[b])

    def wait_load(k, b):
        pltpu.make_async_copy(
            x_hbm.at[pl.ds(ebase + k * B, B), pl.ds(c * HW, HW)], bufs[b],
            lsems[b]).wait()
        pltpu.make_async_copy(idx_hbm.at[s, k], ibufs[b], lsems[b]).wait()

    def fire_cnt(b):
        pltpu.async_copy(obuf, cnt.at[ibufs[b].at[0]], csem, add=True)

    def wait_cnt(b):
        pltpu.make_async_copy(obuf, cnt.at[ibufs[b].at[0]], csem).wait()

    fire_load(0, 0)
    fire_load(1, 1)
    wait_load(0, 0)
    pltpu.sync_copy(xbuf0, acc.at[ibuf0.at[0]], add=True)
    fire_cnt(0)
    fire_load(2, 0)

    def group(g, carry):
        for b, dk in ((1, 1), (0, 2)):
            k = 2 * g + dk
            wait_load(k, b)
            pltpu.sync_copy(bufs[b], acc.at[ibufs[b].at[0]], add=True)
            wait_cnt(1 - b)
            fire_cnt(b)
            fire_load(k + 2, b)
        return carry

    lax.fori_loop(0, K // 2 - 2, group, 0)

    k = K - 3
    wait_load(k, 1)
    pltpu.sync_copy(xbuf1, acc.at[ibuf1.at[0]], add=True)
    wait_cnt(0)
    fire_cnt(1)
    fire_load(K - 1, 1)
    for k in (K - 2, K - 1):
        b = k % 2
        wait_load(k, b)
        pltpu.sync_copy(bufs[b], acc.at[ibufs[b].at[0]], add=True)
        wait_cnt(1 - b)
        fire_cnt(b)
    wait_cnt((K - 1) % 2)

    plsc.subcore_barrier()

    def emit_chunk(j):
        off = s * RZ + j * ZB
        pltpu.sync_copy(acc.at[pl.ds(off, ZB)], xbuf0.at[pl.ds(0, ZB)])
        pltpu.sync_copy(cnt.at[pl.ds(off, ZB)], cbuf)

        def row(r, carry):
            cv = jnp.maximum(cbuf[r], 1.0)
            for v in range(HW // 16):
                xv = xbuf0[r, pl.ds(v * 16, 16)]
                xbuf0[r, pl.ds(v * 16, 16)] = xv / cv
            return carry

        lax.fori_loop(0, ZB, row, 0)
        pltpu.sync_copy(xbuf0.at[pl.ds(0, ZB)],
                        out_hbm.at[pl.ds(off, ZB), pl.ds(c * HW, HW)])

    for j in range(NZ):
        if j < NZV:
            emit_chunk(j)
        else:
            @pl.when(s < NS - 1)
            def _tail():
                emit_chunk(j)


def kernel(x, index, dim_size):
    del dim_size
    idx3d = index.astype(jnp.int32).reshape(NS, K, 1, B)
    ones = jnp.ones((B, CW), jnp.float32)
    zs = jnp.zeros((ZB, HW), jnp.float32)
    zc = jnp.zeros((ZB, CW), jnp.float32)
    return _sc_mean(x, idx3d, ones, zs, zc)


# speedup vs baseline: 1.0621x; 1.0214x over previous
import functools

import jax
import jax.numpy as jnp
from jax import lax
from jax.experimental import pallas as pl
from jax.experimental.pallas import tpu as pltpu
from jax.experimental.pallas import tpu_sc as plsc

E = 320000
D = 128
N = 10000
NC = 2
NS = 16
HW = D // NC
EW = E // NS
B = 125
K = EW // B
RZ = 640
NP = NS * RZ
CW = 8
ZB = 80
NZ = RZ // ZB
NZV = (N - (NS - 1) * RZ) // ZB

_mesh = plsc.VectorSubcoreMesh(
    core_axis_name="c", subcore_axis_name="s", num_cores=NC, num_subcores=NS
)


@functools.partial(
    pl.kernel,
    out_type=jax.ShapeDtypeStruct((N, D), jnp.float32),
    mesh=_mesh,
    compiler_params=pltpu.CompilerParams(use_tc_tiling_on_sc=False,
                                         needs_layout_passes=False),
    scratch_types=[
        pltpu.VMEM((B, HW), jnp.float32),
        pltpu.VMEM((B, HW), jnp.float32),
        pltpu.VMEM((1, B), jnp.int32),
        pltpu.VMEM((1, B), jnp.int32),
        pltpu.VMEM((B, CW), jnp.float32),
        pltpu.VMEM((ZB, CW), jnp.float32),
        pltpu.SemaphoreType.DMA,
        pltpu.SemaphoreType.DMA,
        pltpu.SemaphoreType.DMA,
        pltpu.SemaphoreType.DMA,
        pltpu.SemaphoreType.DMA,
        pltpu.VMEM_SHARED((NP, HW), jnp.float32),
        pltpu.VMEM_SHARED((NP, CW), jnp.float32),
    ],
)
def _sc_mean(x_hbm, idx_hbm, ones_hbm, zs_hbm, zc_hbm, out_hbm,
             xbuf0, xbuf1, ibuf0, ibuf1, obuf, cbuf,
             lsem0, lsem1, csem, asemA, asemB, acc, cnt):
    c = lax.axis_index("c")
    s = lax.axis_index("s")
    pltpu.sync_copy(zs_hbm, xbuf0.at[pl.ds(0, ZB)])
    pltpu.sync_copy(zc_hbm, cbuf)
    for j in range(NZ):
        pltpu.async_copy(xbuf0.at[pl.ds(0, ZB)],
                         acc.at[pl.ds(s * RZ + j * ZB, ZB)], asemA)
        pltpu.async_copy(cbuf, cnt.at[pl.ds(s * RZ + j * ZB, ZB)], asemB)
    pltpu.sync_copy(ones_hbm, obuf)
    for j in range(NZ):
        pltpu.make_async_copy(xbuf0.at[pl.ds(0, ZB)],
                              acc.at[pl.ds(s * RZ + j * ZB, ZB)], asemA).wait()
        pltpu.make_async_copy(cbuf, cnt.at[pl.ds(s * RZ + j * ZB, ZB)],
                              asemB).wait()
    plsc.subcore_barrier()

    ebase = s * EW
    bufs = (xbuf0, xbuf1)
    ibufs = (ibuf0, ibuf1)
    lsems = (lsem0, lsem1)

    def fire_load(k, b):
        pltpu.async_copy(
            x_hbm.at[pl.ds(ebase + k * B, B), pl.ds(c * HW, HW)], bufs[b],
            lsems[b])
        pltpu.async_copy(idx_hbm.at[s, k], ibufs[b], lsems[b])

    def wait_load(k, b):
        pltpu.make_async_copy(
            x_hbm.at[pl.ds(ebase + k * B, B), pl.ds(c * HW, HW)], bufs[b],
            lsems[b]).wait()
        pltpu.make_async_copy(idx_hbm.at[s, k], ibufs[b], lsems[b]).wait()

    def fire_cnt(b):
        pltpu.async_copy(obuf, cnt.at[ibufs[b].at[0]], csem, add=True)

    def wait_cnt(b):
        pltpu.make_async_copy(obuf, cnt.at[ibufs[b].at[0]], csem).wait()

    fire_load(0, 0)
    fire_load(1, 1)
    wait_load(0, 0)
    pltpu.sync_copy(xbuf0, acc.at[ibuf0.at[0]], add=True)
    fire_cnt(0)
    fire_load(2, 0)

    def group(g, carry):
        for b, dk in ((1, 1), (0, 2)):
            k = 2 * g + dk
            wait_load(k, b)
            pltpu.sync_copy(bufs[b], acc.at[ibufs[b].at[0]], add=True)
            wait_cnt(1 - b)
            fire_cnt(b)
            fire_load(k + 2, b)
        return carry

    lax.fori_loop(0, K // 2 - 2, group, 0)

    k = K - 3
    wait_load(k, 1)
    pltpu.sync_copy(xbuf1, acc.at[ibuf1.at[0]], add=True)
    wait_cnt(0)
    fire_cnt(1)
    fire_load(K - 1, 1)
    for k in (K - 2, K - 1):
        b = k % 2
        wait_load(k, b)
        pltpu.sync_copy(bufs[b], acc.at[ibufs[b].at[0]], add=True)
        wait_cnt(1 - b)
        fire_cnt(b)
    wait_cnt((K - 1) % 2)

    plsc.subcore_barrier()

    def emit_chunk(j):
        off = s * RZ + j * ZB
        pltpu.sync_copy(acc.at[pl.ds(off, ZB)], xbuf0.at[pl.ds(0, ZB)])
        pltpu.sync_copy(cnt.at[pl.ds(off, ZB)], cbuf)

        def row(r, carry):
            rvec = jnp.full((16,), r, dtype=jnp.int32)
            zvec = jnp.zeros((16,), dtype=jnp.int32)
            cv = jnp.maximum(plsc.load_gather(cbuf, [rvec, zvec]), 1.0)
            for v in range(HW // 16):
                xv = xbuf0[r, pl.ds(v * 16, 16)]
                xbuf0[r, pl.ds(v * 16, 16)] = xv / cv
            return carry

        lax.fori_loop(0, ZB, row, 0)
        pltpu.sync_copy(xbuf0.at[pl.ds(0, ZB)],
                        out_hbm.at[pl.ds(off, ZB), pl.ds(c * HW, HW)])

    for j in range(NZ):
        if j < NZV:
            emit_chunk(j)
        else:
            @pl.when(s < NS - 1)
            def _tail():
                emit_chunk(j)


def kernel(x, index, dim_size):
    del dim_size
    idx3d = index.astype(jnp.int32).reshape(NS, K, 1, B)
    ones = jnp.ones((B, CW), jnp.float32)
    zs = jnp.zeros((ZB, HW), jnp.float32)
    zc = jnp.zeros((ZB, CW), jnp.float32)
    return _sc_mean(x, idx3d, ones, zs, zc)
